# baseline (speedup 1.0000x reference)
import functools

import jax
import jax.numpy as jnp
from jax import lax
from jax.experimental import pallas as pl
from jax.experimental.pallas import tpu as pltpu
from jax.experimental.pallas import tpu_sc as plsc

_T, _B, _D, _H = 512, 16, 512, 512
_TB = _T * _B
_NW = 32
_RPW = _TB // _NW
_CH = 64


def _sc_mesh():
    return plsc.VectorSubcoreMesh(core_axis_name="c", subcore_axis_name="s")


def _cumsum16(v, tmp_v, iota):
    for s in (1, 2, 4, 8):
        tmp_v[...] = v
        sh = plsc.load_gather(tmp_v, [jnp.maximum(iota - s, 0)])
        v = v + jnp.where(iota >= s, sh, 0)
    return v


def _build_index_kernel():
    mesh = _sc_mesh()

    @functools.partial(
        pl.kernel,
        mesh=mesh,
        compiler_params=pltpu.CompilerParams(needs_layout_passes=False),
        out_type=[
            jax.ShapeDtypeStruct((_TB,), jnp.int32),
            jax.ShapeDtypeStruct((_TB,), jnp.int32),
            jax.ShapeDtypeStruct((_B,), jnp.int32),
        ],
        scratch_types=[
            pltpu.VMEM((_TB,), jnp.int32),
            pltpu.VMEM((_TB + _B,), jnp.int32),
            pltpu.VMEM((_TB,), jnp.int32),
            pltpu.VMEM((_TB,), jnp.int32),
            pltpu.VMEM((_B,), jnp.int32),
            pltpu.VMEM((_B,), jnp.int32),
        ],
    )
    def idx_kernel(bm_hbm, gidx_hbm, gidx2_hbm, len_hbm,
                   bm_v, perm_v, gidx_v, gidx2_v, len_v, tmp_v):
        wid = lax.axis_index("s") * 2 + lax.axis_index("c")

        @pl.when(wid == 0)
        def _():
            pltpu.sync_copy(bm_hbm, bm_v)
            iota = lax.iota(jnp.int32, _B)
            zeros = jnp.zeros((_B,), jnp.int32)

            def p1(t, carry):
                k0, lacc = carry
                bm = bm_v[pl.ds(t * _B, _B)]
                on = bm > 0
                cs = _cumsum16(bm, tmp_v, iota)
                rank = cs - bm + k0
                plsc.store_scatter(perm_v, [jnp.where(on, rank, _TB + iota)],
                                   iota + t * _B)
                gidx2_v[pl.ds(t * _B, _B)] = jnp.where(on, rank, _TB + iota)
                return (k0 + plsc.all_reduce_population_count(on), lacc + bm)

            total, lvec = lax.fori_loop(0, _T, p1, (zeros, zeros))
            len_v[...] = lvec

            def p2(t, k0):
                on = lvec > t
                pmi = jnp.where(on, 1, 0)
                cs = _cumsum16(pmi, tmp_v, iota)
                rank = jnp.minimum(cs - pmi + k0, _TB - 1)
                g = plsc.load_gather(perm_v, [rank])
                gidx_v[pl.ds(t * _B, _B)] = jnp.where(on, g, iota + t * _B)
                return k0 + plsc.all_reduce_population_count(on)

            lax.fori_loop(0, _T, p2, zeros)

            pltpu.sync_copy(gidx_v, gidx_hbm)
            pltpu.sync_copy(gidx2_v, gidx2_hbm)
            pltpu.sync_copy(len_v, len_hbm)

    return idx_kernel


def _build_gather_kernel():
    mesh = _sc_mesh()
    nch = _RPW // _CH

    @functools.partial(
        pl.kernel,
        mesh=mesh,
        compiler_params=pltpu.CompilerParams(needs_layout_passes=False),
        out_type=jax.ShapeDtypeStruct((_TB, _H), jnp.float32),
        scratch_types=(
            [pltpu.VMEM((_RPW,), jnp.int32)]
            + [pltpu.VMEM((_CH, _H), jnp.float32) for _ in range(3)]
            + [pltpu.SemaphoreType.DMA for _ in range(6)]
        ),
    )
    def gather_kernel(table_hbm, idx_hbm, out_hbm, idx_v, *bufs_sems):
        bufs = bufs_sems[:3]
        gsems = bufs_sems[3:6]
        wsems = bufs_sems[6:9]
        wid = lax.axis_index("s") * 2 + lax.axis_index("c")
        base_w = wid * _RPW
        pltpu.sync_copy(idx_hbm.at[pl.ds(base_w, _RPW)], idx_v)

        def gather(c):
            return pltpu.async_copy(
                table_hbm.at[idx_v.at[pl.ds(c * _CH, _CH)]],
                bufs[c % 3], gsems[c % 3])

        def writeback(c):
            return pltpu.async_copy(
                bufs[c % 3], out_hbm.at[pl.ds(base_w + c * _CH, _CH)],
                wsems[c % 3])

        g = {c: gather(c) for c in range(min(3, nch))}
        w = {}
        for c in range(nch):
            g[c].wait()
            w[c] = writeback(c)
            if c + 3 < nch:
                w[c].wait()
                g[c + 3] = gather(c + 3)
        for c in range(max(0, nch - 3), nch):
            w[c].wait()

    return gather_kernel


_K = 32
_NSC = _T // _K
_TM = _K * _B


def _fused_body(len_ref, cmp_ref, h0_ref, wih_ref, bih_ref, whh_ref, bhh_ref,
                y_ref, hout_ref, gi_s, h_v):
    i = pl.program_id(0)
    lvec = len_ref[...]
    maxl = jnp.max(lvec)

    @pl.when(i == 0)
    def _():
        h_v[...] = h0_ref[...]

    @pl.when(jnp.logical_and((i - 1) * _K < maxl, i <= _NSC))
    def _():
        acc = lax.dot_general(cmp_ref[...].astype(jnp.bfloat16), wih_ref[...],
                              (((1,), (1,)), ((), ())),
                              preferred_element_type=jnp.float32)
        gi_s[pl.ds(i % 2, 1)] = (acc + bih_ref[...])[None]

        h = h_v[...]
        p = (i - 1) % 2
        for k in range(_K):
            t = (i - 1) * _K + k
            gi = gi_s[pl.ds(p, 1), pl.ds(k * _B, _B), :][0]
            gh = lax.dot_general(h.astype(jnp.bfloat16), whh_ref[...],
                                 (((1,), (1,)), ((), ())),
                                 preferred_element_type=jnp.float32) + bhh_ref[...]
            r = jax.nn.sigmoid(gi[:, :_H] + gh[:, :_H])
            z = jax.nn.sigmoid(gi[:, _H:2 * _H] + gh[:, _H:2 * _H])
            n = jnp.tanh(gi[:, 2 * _H:] + r * gh[:, 2 * _H:])
            hn = (1.0 - z) * n + z * h
            valid = jnp.logical_and(lvec > t, t >= 0)
            h = jnp.where(valid, hn, h)
            y_ref[k] = jnp.where(valid, hn, 0.0)
        h_v[...] = h

    @pl.when(i == _NSC + 1)
    def _():
        y_ref[...] = jnp.zeros((_K, _B, _H), jnp.float32)
        hout_ref[...] = h_v[...]


def _gru_fused(lcol, compact, h0, w_ih, b_ih_row, w_hh, b_hh_row):
    return pl.pallas_call(
        _fused_body,
        grid=(_NSC + 2,),
        in_specs=[
            pl.BlockSpec((_B, 1), lambda i: (0, 0)),
            pl.BlockSpec((_TM, _D), lambda i: (jnp.minimum(i, _NSC - 1), 0)),
            pl.BlockSpec((_B, _H), lambda i: (0, 0)),
            pl.BlockSpec((3 * _H, _D), lambda i: (0, 0)),
            pl.BlockSpec((1, 3 * _H), lambda i: (0, 0)),
            pl.BlockSpec((3 * _H, _H), lambda i: (0, 0)),
            pl.BlockSpec((1, 3 * _H), lambda i: (0, 0)),
        ],
        out_specs=[
            pl.BlockSpec((_K, _B, _H),
                         lambda i: (jnp.clip(i - 1, 0, _NSC), 0, 0)),
            pl.BlockSpec((_B, _H), lambda i: (0, 0)),
        ],
        out_shape=[
            jax.ShapeDtypeStruct((_T + _K, _B, _H), jnp.float32),
            jax.ShapeDtypeStruct((_B, _H), jnp.float32),
        ],
        scratch_shapes=[
            pltpu.VMEM((2, _TM, 3 * _H), jnp.float32),
            pltpu.VMEM((_B, _H), jnp.float32),
        ],
    )(lcol, compact, h0, w_ih, b_ih_row, w_hh, b_hh_row)


_idx_call = _build_index_kernel()
_gather_rows = _build_gather_kernel()


def kernel(x, rnn_hxs, batch_mask, W_ih, W_hh, b_ih, b_hh):
    x2d = x.reshape(_TB, _D)
    bm = batch_mask.reshape(_TB).astype(jnp.int32)

    gidx, gidx2, lvec = _idx_call(bm)
    lcol = lvec.reshape(_B, 1)

    compact = _gather_rows(x2d, gidx)
    ypad, h_fin = _gru_fused(lcol, compact, rnn_hxs[0],
                             W_ih.astype(jnp.bfloat16),
                             b_ih.reshape(1, 3 * _H),
                             W_hh.astype(jnp.bfloat16),
                             b_hh.reshape(1, 3 * _H))
    ypad2d = ypad.reshape((_T + _K) * _B, _H)
    scores2d = _gather_rows(ypad2d, gidx2)
    return scores2d.reshape(_T, _B, _H), h_fin[None]


# speedup vs baseline: 1.0793x; 1.0793x over previous
import functools

import jax
import jax.numpy as jnp
from jax import lax
from jax.experimental import pallas as pl
from jax.experimental.pallas import tpu as pltpu
from jax.experimental.pallas import tpu_sc as plsc

_T, _B, _D, _H = 512, 16, 512, 512
_TB = _T * _B
_NW = 32
_RPW = _TB // _NW
_CH = 64


def _sc_mesh():
    return plsc.VectorSubcoreMesh(core_axis_name="c", subcore_axis_name="s")


def _cumsum16(v, tmp_v, iota):
    for s in (1, 2, 4, 8):
        tmp_v[...] = v
        sh = plsc.load_gather(tmp_v, [jnp.maximum(iota - s, 0)])
        v = v + jnp.where(iota >= s, sh, 0)
    return v


def _build_index_kernel():
    mesh = _sc_mesh()

    @functools.partial(
        pl.kernel,
        mesh=mesh,
        compiler_params=pltpu.CompilerParams(needs_layout_passes=False),
        out_type=[
            jax.ShapeDtypeStruct((_TB,), jnp.int32),
            jax.ShapeDtypeStruct((_TB,), jnp.int32),
            jax.ShapeDtypeStruct((_B,), jnp.int32),
        ],
        scratch_types=[
            pltpu.VMEM((_TB,), jnp.int32),
            pltpu.VMEM((_TB + _B,), jnp.int32),
            pltpu.VMEM((_TB,), jnp.int32),
            pltpu.VMEM((_TB,), jnp.int32),
            pltpu.VMEM((_B,), jnp.int32),
            pltpu.VMEM((_B,), jnp.int32),
        ],
    )
    def idx_kernel(bm_hbm, gidx_hbm, gidx2_hbm, len_hbm,
                   bm_v, perm_v, gidx_v, gidx2_v, len_v, tmp_v):
        wid = lax.axis_index("s") * 2 + lax.axis_index("c")

        @pl.when(wid == 0)
        def _():
            pltpu.sync_copy(bm_hbm, bm_v)
            iota = lax.iota(jnp.int32, _B)
            zeros = jnp.zeros((_B,), jnp.int32)

            def p1(t, carry):
                k0, lacc = carry
                bm = bm_v[pl.ds(t * _B, _B)]
                on = bm > 0
                cs = _cumsum16(bm, tmp_v, iota)
                rank = cs - bm + k0
                plsc.store_scatter(perm_v, [jnp.where(on, rank, _TB + iota)],
                                   iota + t * _B)
                pad = _TB + ((iota + t * _B) & (_K * _B - 1))
                gidx2_v[pl.ds(t * _B, _B)] = jnp.where(on, rank, pad)
                return (k0 + plsc.all_reduce_population_count(on), lacc + bm)

            total, lvec = lax.fori_loop(0, _T, p1, (zeros, zeros))
            len_v[...] = lvec

            def p2(t, k0):
                on = lvec > t
                pmi = jnp.where(on, 1, 0)
                cs = _cumsum16(pmi, tmp_v, iota)
                rank = jnp.minimum(cs - pmi + k0, _TB - 1)
                g = plsc.load_gather(perm_v, [rank])
                gidx_v[pl.ds(t * _B, _B)] = jnp.where(on, g, iota + t * _B)
                return k0 + plsc.all_reduce_population_count(on)

            lax.fori_loop(0, _T, p2, zeros)

            pltpu.sync_copy(gidx_v, gidx_hbm)
            pltpu.sync_copy(gidx2_v, gidx2_hbm)
            pltpu.sync_copy(len_v, len_hbm)

    return idx_kernel


def _build_gather_kernel():
    mesh = _sc_mesh()
    nch = _RPW // _CH

    @functools.partial(
        pl.kernel,
        mesh=mesh,
        compiler_params=pltpu.CompilerParams(needs_layout_passes=False),
        out_type=jax.ShapeDtypeStruct((_TB, _H), jnp.float32),
        scratch_types=(
            [pltpu.VMEM((_RPW,), jnp.int32)]
            + [pltpu.VMEM((_CH, _H), jnp.float32) for _ in range(3)]
            + [pltpu.SemaphoreType.DMA for _ in range(6)]
        ),
    )
    def gather_kernel(table_hbm, idx_hbm, out_hbm, idx_v, *bufs_sems):
        bufs = bufs_sems[:3]
        gsems = bufs_sems[3:6]
        wsems = bufs_sems[6:9]
        wid = lax.axis_index("s") * 2 + lax.axis_index("c")
        base_w = wid * _RPW
        pltpu.sync_copy(idx_hbm.at[pl.ds(base_w, _RPW)], idx_v)

        def gather(c):
            return pltpu.async_copy(
                table_hbm.at[idx_v.at[pl.ds(c * _CH, _CH)]],
                bufs[c % 3], gsems[c % 3])

        def writeback(c):
            return pltpu.async_copy(
                bufs[c % 3], out_hbm.at[pl.ds(base_w + c * _CH, _CH)],
                wsems[c % 3])

        g = {c: gather(c) for c in range(min(3, nch))}
        w = {}
        for c in range(nch):
            g[c].wait()
            w[c] = writeback(c)
            if c + 3 < nch:
                w[c].wait()
                g[c + 3] = gather(c + 3)
        for c in range(max(0, nch - 3), nch):
            w[c].wait()

    return gather_kernel


_K = 32
_NSC = _T // _K
_TM = _K * _B


def _fused_body(len_ref, cmp_ref, h0_ref, wih_ref, bih_ref, whh_ref, bhh_ref,
                y_ref, hout_ref, gi_s, h_v):
    i = pl.program_id(0)
    lvec = len_ref[...]
    maxl = jnp.max(lvec)

    @pl.when(i == 0)
    def _():
        h_v[...] = h0_ref[...]

    @pl.when(jnp.logical_and((i - 1) * _K < maxl, i <= _NSC))
    def _():
        acc = lax.dot_general(cmp_ref[...].astype(jnp.bfloat16), wih_ref[...],
                              (((1,), (1,)), ((), ())),
                              preferred_element_type=jnp.float32)
        gi_s[pl.ds(i % 2, 1)] = (acc + bih_ref[...])[None]

        h = h_v[...]
        p = (i - 1) % 2
        for k in range(_K):
            t = (i - 1) * _K + k
            gi = gi_s[pl.ds(p, 1), pl.ds(k * _B, _B), :][0]
            gh = lax.dot_general(h.astype(jnp.bfloat16), whh_ref[...],
                                 (((1,), (1,)), ((), ())),
                                 preferred_element_type=jnp.float32) + bhh_ref[...]
            r = jax.nn.sigmoid(gi[:, :_H] + gh[:, :_H])
            z = jax.nn.sigmoid(gi[:, _H:2 * _H] + gh[:, _H:2 * _H])
            n = jnp.tanh(gi[:, 2 * _H:] + r * gh[:, 2 * _H:])
            hn = (1.0 - z) * n + z * h
            valid = jnp.logical_and(lvec > t, t >= 0)
            h = jnp.where(valid, hn, h)
            y_ref[k] = jnp.where(valid, hn, 0.0)
        h_v[...] = h

    @pl.when(i == _NSC + 1)
    def _():
        y_ref[...] = jnp.zeros((_K, _B, _H), jnp.float32)
        hout_ref[...] = h_v[...]


def _gru_fused(lcol, compact, h0, w_ih, b_ih_row, w_hh, b_hh_row):
    return pl.pallas_call(
        _fused_body,
        grid=(_NSC + 2,),
        in_specs=[
            pl.BlockSpec((_B, 1), lambda i: (0, 0)),
            pl.BlockSpec((_TM, _D), lambda i: (jnp.minimum(i, _NSC - 1), 0)),
            pl.BlockSpec((_B, _H), lambda i: (0, 0)),
            pl.BlockSpec((3 * _H, _D), lambda i: (0, 0)),
            pl.BlockSpec((1, 3 * _H), lambda i: (0, 0)),
            pl.BlockSpec((3 * _H, _H), lambda i: (0, 0)),
            pl.BlockSpec((1, 3 * _H), lambda i: (0, 0)),
        ],
        out_specs=[
            pl.BlockSpec((_K, _B, _H),
                         lambda i: (jnp.clip(i - 1, 0, _NSC), 0, 0)),
            pl.BlockSpec((_B, _H), lambda i: (0, 0)),
        ],
        out_shape=[
            jax.ShapeDtypeStruct((_T + _K, _B, _H), jnp.float32),
            jax.ShapeDtypeStruct((_B, _H), jnp.float32),
        ],
        scratch_shapes=[
            pltpu.VMEM((2, _TM, 3 * _H), jnp.float32),
            pltpu.VMEM((_B, _H), jnp.float32),
        ],
    )(lcol, compact, h0, w_ih, b_ih_row, w_hh, b_hh_row)


_idx_call = _build_index_kernel()
_gather_rows = _build_gather_kernel()


def kernel(x, rnn_hxs, batch_mask, W_ih, W_hh, b_ih, b_hh):
    x2d = x.reshape(_TB, _D)
    bm = batch_mask.reshape(_TB).astype(jnp.int32)

    gidx, gidx2, lvec = _idx_call(bm)
    lcol = lvec.reshape(_B, 1)

    compact = _gather_rows(x2d, gidx)
    ypad, h_fin = _gru_fused(lcol, compact, rnn_hxs[0],
                             W_ih.astype(jnp.bfloat16),
                             b_ih.reshape(1, 3 * _H),
                             W_hh.astype(jnp.bfloat16),
                             b_hh.reshape(1, 3 * _H))
    ypad2d = ypad.reshape((_T + _K) * _B, _H)
    scores2d = _gather_rows(ypad2d, gidx2)
    return scores2d.reshape(_T, _B, _H), h_fin[None]


# speedup vs baseline: 1.0895x; 1.0094x over previous
import functools

import jax
import jax.numpy as jnp
from jax import lax
from jax.experimental import pallas as pl
from jax.experimental.pallas import tpu as pltpu
from jax.experimental.pallas import tpu_sc as plsc

_T, _B, _D, _H = 512, 16, 512, 512
_TB = _T * _B
_NW = 32
_RPW = _TB // _NW
_CH = 64


def _sc_mesh():
    return plsc.VectorSubcoreMesh(core_axis_name="c", subcore_axis_name="s")


def _cumsum16(v, tmp_v, iota):
    for s in (1, 2, 4, 8):
        tmp_v[...] = v
        sh = plsc.load_gather(tmp_v, [jnp.maximum(iota - s, 0)])
        v = v + jnp.where(iota >= s, sh, 0)
    return v


def _build_index_gather_kernel():
    mesh = _sc_mesh()

    @functools.partial(
        pl.kernel,
        mesh=mesh,
        compiler_params=pltpu.CompilerParams(needs_layout_passes=False),
        out_type=[
            jax.ShapeDtypeStruct((_TB, _H), jnp.float32),
            jax.ShapeDtypeStruct((_TB,), jnp.int32),
            jax.ShapeDtypeStruct((_B,), jnp.int32),
        ],
        scratch_types=[
            pltpu.VMEM((_TB,), jnp.int32),
            pltpu.VMEM((_TB + _B,), jnp.int32),
            pltpu.VMEM((_TB,), jnp.int32),
            pltpu.VMEM((_TB,), jnp.int32),
            pltpu.VMEM((_B,), jnp.int32),
            pltpu.VMEM((_B,), jnp.int32),
            pltpu.VMEM_SHARED((_TB,), jnp.int32),
            pltpu.VMEM((_RPW,), jnp.int32),
        ] + [pltpu.VMEM((_CH, _H), jnp.float32) for _ in range(2)]
          + [pltpu.SemaphoreType.DMA for _ in range(4)],
    )
    def idx_kernel(bm_hbm, x_hbm, compact_hbm, gidx2_hbm, len_hbm,
                   bm_v, perm_v, gidx_v, gidx2_v, len_v, tmp_v,
                   gidx_sh, lidx_v, *bufs_sems):
        sid = lax.axis_index("s")
        cid = lax.axis_index("c")

        @pl.when(sid == 0)
        def _():
            pltpu.sync_copy(bm_hbm, bm_v)
            iota = lax.iota(jnp.int32, _B)
            zeros = jnp.zeros((_B,), jnp.int32)

            def p1(t, carry):
                k0, lacc = carry
                bm = bm_v[pl.ds(t * _B, _B)]
                on = bm > 0
                cs = _cumsum16(bm, tmp_v, iota)
                rank = cs - bm + k0
                plsc.store_scatter(perm_v, [jnp.where(on, rank, _TB + iota)],
                                   iota + t * _B)
                pad = _TB + ((iota + t * _B) & (_K * _B - 1))
                gidx2_v[pl.ds(t * _B, _B)] = jnp.where(on, rank, pad)
                return (k0 + plsc.all_reduce_population_count(on), lacc + bm)

            total, lvec = lax.fori_loop(0, _T, p1, (zeros, zeros))
            len_v[...] = lvec

            def p2(t, k0):
                on = lvec > t
                pmi = jnp.where(on, 1, 0)
                cs = _cumsum16(pmi, tmp_v, iota)
                rank = jnp.minimum(cs - pmi + k0, _TB - 1)
                g = plsc.load_gather(perm_v, [rank])
                gidx_v[pl.ds(t * _B, _B)] = jnp.where(on, g, iota + t * _B)
                return k0 + plsc.all_reduce_population_count(on)

            lax.fori_loop(0, _T, p2, zeros)

            pltpu.sync_copy(gidx_v, gidx_sh)

            @pl.when(cid == 0)
            def _():
                pltpu.sync_copy(gidx2_v, gidx2_hbm)
                pltpu.sync_copy(len_v, len_hbm)

        plsc.subcore_barrier()

        bufs = bufs_sems[:2]
        gsems = bufs_sems[2:4]
        wsems = bufs_sems[4:6]
        wid = sid * 2 + cid
        base_w = wid * _RPW
        pltpu.sync_copy(gidx_sh.at[pl.ds(base_w, _RPW)], lidx_v)

        def gather(c):
            return pltpu.async_copy(
                x_hbm.at[lidx_v.at[pl.ds(c * _CH, _CH)]],
                bufs[c % 2], gsems[c % 2])

        def writeback(c):
            return pltpu.async_copy(
                bufs[c % 2], compact_hbm.at[pl.ds(base_w + c * _CH, _CH)],
                wsems[c % 2])

        nch = _RPW // _CH
        g = {c: gather(c) for c in range(min(2, nch))}
        w = {}
        for c in range(nch):
            g[c].wait()
            w[c] = writeback(c)
            if c + 2 < nch:
                w[c].wait()
                g[c + 2] = gather(c + 2)
        for c in range(max(0, nch - 2), nch):
            w[c].wait()

    return idx_kernel


def _build_gather_kernel():
    mesh = _sc_mesh()
    nch = _RPW // _CH

    @functools.partial(
        pl.kernel,
        mesh=mesh,
        compiler_params=pltpu.CompilerParams(needs_layout_passes=False),
        out_type=jax.ShapeDtypeStruct((_TB, _H), jnp.float32),
        scratch_types=(
            [pltpu.VMEM((_RPW,), jnp.int32)]
            + [pltpu.VMEM((_CH, _H), jnp.float32) for _ in range(3)]
            + [pltpu.SemaphoreType.DMA for _ in range(6)]
        ),
    )
    def gather_kernel(table_hbm, idx_hbm, out_hbm, idx_v, *bufs_sems):
        bufs = bufs_sems[:3]
        gsems = bufs_sems[3:6]
        wsems = bufs_sems[6:9]
        wid = lax.axis_index("s") * 2 + lax.axis_index("c")
        base_w = wid * _RPW
        pltpu.sync_copy(idx_hbm.at[pl.ds(base_w, _RPW)], idx_v)

        def gather(c):
            return pltpu.async_copy(
                table_hbm.at[idx_v.at[pl.ds(c * _CH, _CH)]],
                bufs[c % 3], gsems[c % 3])

        def writeback(c):
            return pltpu.async_copy(
                bufs[c % 3], out_hbm.at[pl.ds(base_w + c * _CH, _CH)],
                wsems[c % 3])

        g = {c: gather(c) for c in range(min(3, nch))}
        w = {}
        for c in range(nch):
            g[c].wait()
            w[c] = writeback(c)
            if c + 3 < nch:
                w[c].wait()
                g[c + 3] = gather(c + 3)
        for c in range(max(0, nch - 3), nch):
            w[c].wait()

    return gather_kernel


_K = 32
_NSC = _T // _K
_TM = _K * _B


def _fused_body(len_ref, cmp_ref, h0_ref, wih_ref, bih_ref, whh_ref, bhh_ref,
                y_ref, hout_ref, gi_s, h_v):
    i = pl.program_id(0)
    lvec = len_ref[...]
    maxl = jnp.max(lvec)

    @pl.when(i == 0)
    def _():
        h_v[...] = h0_ref[...]

    @pl.when(jnp.logical_and((i - 1) * _K < maxl, i <= _NSC))
    def _():
        acc = lax.dot_general(cmp_ref[...].astype(jnp.bfloat16), wih_ref[...],
                              (((1,), (1,)), ((), ())),
                              preferred_element_type=jnp.float32)
        gi_s[pl.ds(i % 2, 1)] = (acc + bih_ref[...])[None]

        h = h_v[...]
        p = (i - 1) % 2
        for k in range(_K):
            t = (i - 1) * _K + k
            gi = gi_s[pl.ds(p, 1), pl.ds(k * _B, _B), :][0]
            gh = lax.dot_general(h.astype(jnp.bfloat16), whh_ref[...],
                                 (((1,), (1,)), ((), ())),
                                 preferred_element_type=jnp.float32) + bhh_ref[...]
            r = jax.nn.sigmoid(gi[:, :_H] + gh[:, :_H])
            z = jax.nn.sigmoid(gi[:, _H:2 * _H] + gh[:, _H:2 * _H])
            n = jnp.tanh(gi[:, 2 * _H:] + r * gh[:, 2 * _H:])
            hn = (1.0 - z) * n + z * h
            valid = jnp.logical_and(lvec > t, t >= 0)
            h = jnp.where(valid, hn, h)
            y_ref[k] = jnp.where(valid, hn, 0.0)
        h_v[...] = h

    @pl.when(i == _NSC + 1)
    def _():
        y_ref[...] = jnp.zeros((_K, _B, _H), jnp.float32)
        hout_ref[...] = h_v[...]


def _gru_fused(lcol, compact, h0, w_ih, b_ih_row, w_hh, b_hh_row):
    return pl.pallas_call(
        _fused_body,
        grid=(_NSC + 2,),
        in_specs=[
            pl.BlockSpec((_B, 1), lambda i: (0, 0)),
            pl.BlockSpec((_TM, _D), lambda i: (jnp.minimum(i, _NSC - 1), 0)),
            pl.BlockSpec((_B, _H), lambda i: (0, 0)),
            pl.BlockSpec((3 * _H, _D), lambda i: (0, 0)),
            pl.BlockSpec((1, 3 * _H), lambda i: (0, 0)),
            pl.BlockSpec((3 * _H, _H), lambda i: (0, 0)),
            pl.BlockSpec((1, 3 * _H), lambda i: (0, 0)),
        ],
        out_specs=[
            pl.BlockSpec((_K, _B, _H),
                         lambda i: (jnp.clip(i - 1, 0, _NSC), 0, 0)),
            pl.BlockSpec((_B, _H), lambda i: (0, 0)),
        ],
        out_shape=[
            jax.ShapeDtypeStruct((_T + _K, _B, _H), jnp.float32),
            jax.ShapeDtypeStruct((_B, _H), jnp.float32),
        ],
        scratch_shapes=[
            pltpu.VMEM((2, _TM, 3 * _H), jnp.float32),
            pltpu.VMEM((_B, _H), jnp.float32),
        ],
    )(lcol, compact, h0, w_ih, b_ih_row, w_hh, b_hh_row)


_idx_gather_call = _build_index_gather_kernel()
_gather_rows = _build_gather_kernel()


def kernel(x, rnn_hxs, batch_mask, W_ih, W_hh, b_ih, b_hh):
    x2d = x.reshape(_TB, _D)
    bm = batch_mask.reshape(_TB).astype(jnp.int32)

    compact, gidx2, lvec = _idx_gather_call(bm, x2d)
    lcol = lvec.reshape(_B, 1)
    ypad, h_fin = _gru_fused(lcol, compact, rnn_hxs[0],
                             W_ih.astype(jnp.bfloat16),
                             b_ih.reshape(1, 3 * _H),
                             W_hh.astype(jnp.bfloat16),
                             b_hh.reshape(1, 3 * _H))
    ypad2d = ypad.reshape((_T + _K) * _B, _H)
    scores2d = _gather_rows(ypad2d, gidx2)
    return scores2d.reshape(_T, _B, _H), h_fin[None]
